# baseline (speedup 1.0000x reference)
import functools

import jax
import jax.numpy as jnp
from jax import lax
from jax.experimental import pallas as pl
from jax.experimental.pallas import tpu as pltpu
from jax.experimental.pallas import tpu_sc as plsc

N = 10000
NP = 10240
E = 320000
G = 128
DIN = 128
H = 64
NC, NS, L = 2, 16, 16
NW = NC * NS
CH = 80
EP = NW * CH * 128
CHUNK = 128
ROWS_PER_TILE = NP // NS

_EPS = 1e-5
_f32 = jnp.float32


def _mesh():
  return plsc.VectorSubcoreMesh(core_axis_name="c", subcore_axis_name="s",
                                num_cores=NC, num_subcores=NS)


def _fill_const(buf, width, value):
  @pl.loop(0, CHUNK)
  def _fill(i):
    for kk in range(width // L):
      buf[i, pl.ds(kk * L, L)] = jnp.full((L,), value, _f32)


@functools.cache
def _deg_kernel():
  @functools.partial(
      pl.kernel,
      out_type=[jax.ShapeDtypeStruct((NC, NP, 16), _f32),
                jax.ShapeDtypeStruct((NC, NP, 16), _f32)],
      mesh=_mesh(),
      scratch_types=[
          pltpu.VMEM((CH, CHUNK), jnp.int32),
          pltpu.VMEM((CH, CHUNK), jnp.int32),
          pltpu.VMEM((CHUNK, 16), _f32),
          pltpu.VMEM_SHARED((NP, 16), _f32),
          pltpu.VMEM_SHARED((NP, 16), _f32),
      ],
      compiler_params=pltpu.CompilerParams(use_tc_tiling_on_sc=False),
  )
  def k(src_hbm, dst_hbm, zeros_hbm, dego_hbm, degi_hbm,
        idx_s, idx_d, ones_v, dego_sh, degi_sh):
    c = lax.axis_index("c")
    s = lax.axis_index("s")
    w = s * NC + c
    r0 = s * ROWS_PER_TILE

    _fill_const(ones_v, 16, 1.0)

    pltpu.sync_copy(zeros_hbm.at[pl.ds(r0, ROWS_PER_TILE)],
                    dego_sh.at[pl.ds(r0, ROWS_PER_TILE)])
    pltpu.sync_copy(zeros_hbm.at[pl.ds(r0, ROWS_PER_TILE)],
                    degi_sh.at[pl.ds(r0, ROWS_PER_TILE)])
    plsc.subcore_barrier()

    pltpu.sync_copy(src_hbm.at[pl.ds(w * CH, CH)], idx_s)
    pltpu.sync_copy(dst_hbm.at[pl.ds(w * CH, CH)], idx_d)

    @pl.loop(0, CH)
    def _scatter(j):
      pltpu.sync_copy(ones_v, dego_sh.at[idx_s.at[j]], add=True)
      pltpu.sync_copy(ones_v, degi_sh.at[idx_d.at[j]], add=True)

    plsc.subcore_barrier()
    pltpu.sync_copy(dego_sh.at[pl.ds(r0, ROWS_PER_TILE)],
                    dego_hbm.at[c, pl.ds(r0, ROWS_PER_TILE)])
    pltpu.sync_copy(degi_sh.at[pl.ds(r0, ROWS_PER_TILE)],
                    degi_hbm.at[c, pl.ds(r0, ROWS_PER_TILE)])

  return k


def _sc_degrees(srcp, dstp, zeros16):
  return _deg_kernel()(srcp, dstp, zeros16)


@functools.cache
def _agg_kernel():
  @functools.partial(
      pl.kernel,
      out_type=jax.ShapeDtypeStruct((NC, NP, H), _f32),
      mesh=_mesh(),
      scratch_types=[
          pltpu.VMEM((CH, CHUNK), jnp.int32),
          pltpu.VMEM((CH, CHUNK), jnp.int32),
          pltpu.VMEM((CHUNK, H), _f32),
          pltpu.VMEM((CHUNK, H), _f32),
          pltpu.VMEM_SHARED((NP, H), _f32),
          pltpu.VMEM_SHARED((NP, H), _f32),
          pltpu.SemaphoreType.DMA,
          pltpu.SemaphoreType.DMA,
      ],
      compiler_params=pltpu.CompilerParams(use_tc_tiling_on_sc=False),
  )
  def k(ytab_hbm, src_hbm, dst_hbm, zeros_hbm, out_hbm,
        idx_s, idx_d, rows_a, rows_b, ytab_sh, agg_sh, sem_a, sem_b):
    c = lax.axis_index("c")
    s = lax.axis_index("s")
    w = s * NC + c
    r0 = s * ROWS_PER_TILE

    pltpu.sync_copy(zeros_hbm.at[pl.ds(r0, ROWS_PER_TILE)],
                    agg_sh.at[pl.ds(r0, ROWS_PER_TILE)])
    pltpu.sync_copy(ytab_hbm.at[pl.ds(r0, ROWS_PER_TILE)],
                    ytab_sh.at[pl.ds(r0, ROWS_PER_TILE)])
    plsc.subcore_barrier()

    pltpu.sync_copy(src_hbm.at[pl.ds(w * CH, CH)], idx_s)
    pltpu.sync_copy(dst_hbm.at[pl.ds(w * CH, CH)], idx_d)

    def start(j, rows, sem):
      pltpu.async_copy(ytab_sh.at[idx_s.at[j]], rows, sem)

    def wait(j, rows, sem):
      pltpu.make_async_copy(ytab_sh.at[idx_s.at[j]], rows, sem).wait()

    start(0, rows_a, sem_a)

    @pl.loop(0, CH // 2)
    def _edge_chunk(jj):
      j0 = jj * 2
      start(j0 + 1, rows_b, sem_b)
      wait(j0, rows_a, sem_a)
      pltpu.sync_copy(rows_a, agg_sh.at[idx_d.at[j0]], add=True)

      @pl.when(jj < CH // 2 - 1)
      def _next():
        start(j0 + 2, rows_a, sem_a)

      wait(j0 + 1, rows_b, sem_b)
      pltpu.sync_copy(rows_b, agg_sh.at[idx_d.at[j0 + 1]], add=True)

    plsc.subcore_barrier()
    pltpu.sync_copy(agg_sh.at[pl.ds(r0, ROWS_PER_TILE)],
                    out_hbm.at[c, pl.ds(r0, ROWS_PER_TILE)])

  return k


def _sc_aggregate(ytab, srcp, dstp, zeros64):
  return _agg_kernel()(ytab, srcp, dstp, zeros64)


BLK = 640
NBLK = NP // BLK


def _norm_from(p0, p1):
  deg = p0[:, :1] + p1[:, :1]
  return jnp.where(deg > 0, lax.rsqrt(jnp.maximum(deg, 1.0)), 0.0)


def _tc_layer0(x, dgo0, dgo1, W0, Rw0, Rb0):
  def body(x_ref, o0_ref, o1_ref, w_ref, rw_ref, rb_ref, y_ref, res_ref):
    xb = x_ref[...]
    norm_o = _norm_from(o0_ref[...], o1_ref[...])
    y_ref[...] = jnp.dot(xb * norm_o, w_ref[...],
                         preferred_element_type=_f32)
    res_ref[...] = jax.nn.relu(
        jnp.dot(xb, rw_ref[...], preferred_element_type=_f32) + rb_ref[...])

  return pl.pallas_call(
      body,
      grid=(NBLK,),
      in_specs=[
          pl.BlockSpec((BLK, DIN), lambda k: (k, 0)),
          pl.BlockSpec((BLK, 16), lambda k: (k, 0)),
          pl.BlockSpec((BLK, 16), lambda k: (k, 0)),
          pl.BlockSpec((DIN, H), lambda k: (0, 0)),
          pl.BlockSpec((DIN, H), lambda k: (0, 0)),
          pl.BlockSpec((1, H), lambda k: (0, 0)),
      ],
      out_specs=[
          pl.BlockSpec((BLK, H), lambda k: (k, 0)),
          pl.BlockSpec((BLK, H), lambda k: (k, 0)),
      ],
      out_shape=[jax.ShapeDtypeStruct((NP, H), _f32),
                 jax.ShapeDtypeStruct((NP, H), _f32)],
  )(x, dgo0, dgo1, W0, Rw0, Rb0)


def _tc_mid(p0, p1, res_prev, dgo0, dgo1, dgi0, dgi1,
            b_prev, g_prev, beta_prev, m_prev, v_prev, Wn, Rwn, Rbn):
  def body(p0_ref, p1_ref, rp_ref, o0_ref, o1_ref, i0_ref, i1_ref,
           b_ref, g_ref, be_ref, m_ref, v_ref, w_ref, rw_ref, rb_ref,
           y_ref, res_ref):
    norm_i = _norm_from(i0_ref[...], i1_ref[...])
    h = jax.nn.relu(norm_i * (p0_ref[...] + p1_ref[...]) + b_ref[...])
    h = h + rp_ref[...]
    h = (h - m_ref[...]) * lax.rsqrt(v_ref[...] + _EPS) * g_ref[...] \
        + be_ref[...]
    norm_o = _norm_from(o0_ref[...], o1_ref[...])
    y_ref[...] = jnp.dot(h * norm_o, w_ref[...],
                         preferred_element_type=_f32)
    res_ref[...] = jax.nn.relu(
        jnp.dot(h, rw_ref[...], preferred_element_type=_f32) + rb_ref[...])

  row = lambda k: (k, 0)
  fix = lambda k: (0, 0)
  return pl.pallas_call(
      body,
      grid=(NBLK,),
      in_specs=[
          pl.BlockSpec((BLK, H), row), pl.BlockSpec((BLK, H), row),
          pl.BlockSpec((BLK, H), row),
          pl.BlockSpec((BLK, 16), row), pl.BlockSpec((BLK, 16), row),
          pl.BlockSpec((BLK, 16), row), pl.BlockSpec((BLK, 16), row),
          pl.BlockSpec((1, H), fix), pl.BlockSpec((1, H), fix),
          pl.BlockSpec((1, H), fix), pl.BlockSpec((1, H), fix),
          pl.BlockSpec((1, H), fix),
          pl.BlockSpec((H, H), fix), pl.BlockSpec((H, H), fix),
          pl.BlockSpec((1, H), fix),
      ],
      out_specs=[
          pl.BlockSpec((BLK, H), row),
          pl.BlockSpec((BLK, H), row),
      ],
      out_shape=[jax.ShapeDtypeStruct((NP, H), _f32),
                 jax.ShapeDtypeStruct((NP, H), _f32)],
  )(p0, p1, res_prev, dgo0, dgo1, dgi0, dgi1,
    b_prev, g_prev, beta_prev, m_prev, v_prev, Wn, Rwn, Rbn)


def _tc_final(p0, p1, res_prev, dgi0, dgi1,
              b2, g2, beta2, m2, v2, gid3, Wc1, bc1, Wc2, bc2):
  def body(p0_ref, p1_ref, rp_ref, i0_ref, i1_ref,
           b_ref, g_ref, be_ref, m_ref, v_ref, gid_ref,
           wc1_ref, bc1_ref, wc2_ref, bc2_ref, pooled_ref, out_ref):
    k = pl.program_id(0)
    norm_i = _norm_from(i0_ref[...], i1_ref[...])
    h = jax.nn.relu(norm_i * (p0_ref[...] + p1_ref[...]) + b_ref[...])
    h = h + rp_ref[...]
    h = (h - m_ref[...]) * lax.rsqrt(v_ref[...] + _EPS) * g_ref[...] \
        + be_ref[...]
    gid = gid_ref[0]
    seg = lax.broadcasted_iota(jnp.int32, (G, BLK), 0)
    mask = (seg == gid).astype(_f32)
    contrib = jnp.dot(mask, h, preferred_element_type=_f32)

    @pl.when(k == 0)
    def _init():
      pooled_ref[...] = contrib

    @pl.when(k > 0)
    def _acc():
      pooled_ref[...] += contrib

    @pl.when(k == NBLK - 1)
    def _head():
      pooled = pooled_ref[...]
      hid = jax.nn.relu(
          jnp.dot(pooled, wc1_ref[...], preferred_element_type=_f32)
          + bc1_ref[...])
      out_ref[...] = jnp.dot(hid, wc2_ref[...],
                             preferred_element_type=_f32) + bc2_ref[...]

  row = lambda k: (k, 0)
  fix = lambda k: (0, 0)
  return pl.pallas_call(
      body,
      grid=(NBLK,),
      in_specs=[
          pl.BlockSpec((BLK, H), row), pl.BlockSpec((BLK, H), row),
          pl.BlockSpec((BLK, H), row),
          pl.BlockSpec((BLK, 16), row), pl.BlockSpec((BLK, 16), row),
          pl.BlockSpec((1, H), fix), pl.BlockSpec((1, H), fix),
          pl.BlockSpec((1, H), fix), pl.BlockSpec((1, H), fix),
          pl.BlockSpec((1, H), fix),
          pl.BlockSpec((1, 1, BLK), lambda k: (k, 0, 0)),
          pl.BlockSpec((H, G), fix), pl.BlockSpec((1, G), fix),
          pl.BlockSpec((G, 2), fix), pl.BlockSpec((1, 2), fix),
      ],
      out_specs=[
          pl.BlockSpec((G, H), fix),
          pl.BlockSpec((G, 2), fix),
      ],
      out_shape=[jax.ShapeDtypeStruct((G, H), _f32),
                 jax.ShapeDtypeStruct((G, 2), _f32)],
  )(p0, p1, res_prev, dgi0, dgi1,
    b2, g2, beta2, m2, v2, gid3, Wc1, bc1, Wc2, bc2)[1]


def kernel(node_feats, edge_index, graph_ids,
           W0, b0, Rw0, Rb0, g0, beta0, m0, v0,
           W1, b1, Rw1, Rb1, g1, beta1, m1, v1,
           W2, b2, Rw2, Rb2, g2, beta2, m2, v2,
           Wc1, bc1, Wc2, bc2):
  x = jnp.pad(node_feats, ((0, NP - N), (0, 0)))
  pad_e = EP - E
  srcp = jnp.concatenate(
      [edge_index[0], jnp.full((pad_e,), N, jnp.int32)]).reshape(NW * CH, CHUNK)
  dstp = jnp.concatenate(
      [edge_index[1], jnp.full((pad_e,), N, jnp.int32)]).reshape(NW * CH, CHUNK)
  gid3 = jnp.pad(graph_ids, (0, NP - N),
                 constant_values=G + 1).reshape(NBLK, 1, BLK)
  zeros64 = jnp.zeros((NP, H), _f32)
  zeros16 = jnp.zeros((NP, 16), _f32)
  as_row = lambda t: t.reshape(1, -1)

  dego, degi = _sc_degrees(srcp, dstp, zeros16)
  dgo0, dgo1 = dego[0], dego[1]
  dgi0, dgi1 = degi[0], degi[1]

  y0, res0 = _tc_layer0(x, dgo0, dgo1, W0, Rw0, as_row(Rb0))
  p = _sc_aggregate(y0, srcp, dstp, zeros64)

  y1, res1 = _tc_mid(p[0], p[1], res0, dgo0, dgo1, dgi0, dgi1,
                     as_row(b0), as_row(g0), as_row(beta0), as_row(m0),
                     as_row(v0), W1, Rw1, as_row(Rb1))
  p = _sc_aggregate(y1, srcp, dstp, zeros64)

  y2, res2 = _tc_mid(p[0], p[1], res1, dgo0, dgo1, dgi0, dgi1,
                     as_row(b1), as_row(g1), as_row(beta1), as_row(m1),
                     as_row(v1), W2, Rw2, as_row(Rb2))
  p = _sc_aggregate(y2, srcp, dstp, zeros64)

  return _tc_final(p[0], p[1], res2, dgi0, dgi1,
                   as_row(b2), as_row(g2), as_row(beta2), as_row(m2),
                   as_row(v2), gid3, Wc1, bc1.reshape(1, G), Wc2,
                   bc2.reshape(1, 2))


# speedup vs baseline: 1.0867x; 1.0867x over previous
import functools

import jax
import jax.numpy as jnp
from jax import lax
from jax.experimental import pallas as pl
from jax.experimental.pallas import tpu as pltpu
from jax.experimental.pallas import tpu_sc as plsc

N = 10000
NP = 10240
E = 320000
G = 128
DIN = 128
H = 64
NC, NS, L = 2, 16, 16
NW = NC * NS
CH = 80
EP = NW * CH * 128
CHUNK = 128
ROWS_PER_TILE = NP // NS

_EPS = 1e-5
_f32 = jnp.float32


def _mesh():
  return plsc.VectorSubcoreMesh(core_axis_name="c", subcore_axis_name="s",
                                num_cores=NC, num_subcores=NS)


def _fill_const(buf, width, value):
  @pl.loop(0, CHUNK)
  def _fill(i):
    for kk in range(width // L):
      buf[i, pl.ds(kk * L, L)] = jnp.full((L,), value, _f32)


@functools.cache
def _deg_kernel():
  @functools.partial(
      pl.kernel,
      out_type=[jax.ShapeDtypeStruct((NC, NP, 16), _f32),
                jax.ShapeDtypeStruct((NC, NP, 16), _f32)],
      mesh=_mesh(),
      scratch_types=[
          pltpu.VMEM((CH, CHUNK), jnp.int32),
          pltpu.VMEM((CH, CHUNK), jnp.int32),
          pltpu.VMEM((CHUNK, 16), _f32),
          pltpu.VMEM_SHARED((NP, 16), _f32),
          pltpu.VMEM_SHARED((NP, 16), _f32),
      ],
      compiler_params=pltpu.CompilerParams(use_tc_tiling_on_sc=False),
  )
  def k(src_hbm, dst_hbm, zeros_hbm, dego_hbm, degi_hbm,
        idx_s, idx_d, ones_v, dego_sh, degi_sh):
    c = lax.axis_index("c")
    s = lax.axis_index("s")
    w = s * NC + c
    r0 = s * ROWS_PER_TILE

    _fill_const(ones_v, 16, 1.0)

    pltpu.sync_copy(zeros_hbm.at[pl.ds(r0, ROWS_PER_TILE)],
                    dego_sh.at[pl.ds(r0, ROWS_PER_TILE)])
    pltpu.sync_copy(zeros_hbm.at[pl.ds(r0, ROWS_PER_TILE)],
                    degi_sh.at[pl.ds(r0, ROWS_PER_TILE)])
    plsc.subcore_barrier()

    pltpu.sync_copy(src_hbm.at[pl.ds(w * CH, CH)], idx_s)
    pltpu.sync_copy(dst_hbm.at[pl.ds(w * CH, CH)], idx_d)

    @pl.loop(0, CH)
    def _scatter(j):
      pltpu.sync_copy(ones_v, dego_sh.at[idx_s.at[j]], add=True)
      pltpu.sync_copy(ones_v, degi_sh.at[idx_d.at[j]], add=True)

    plsc.subcore_barrier()
    pltpu.sync_copy(dego_sh.at[pl.ds(r0, ROWS_PER_TILE)],
                    dego_hbm.at[c, pl.ds(r0, ROWS_PER_TILE)])
    pltpu.sync_copy(degi_sh.at[pl.ds(r0, ROWS_PER_TILE)],
                    degi_hbm.at[c, pl.ds(r0, ROWS_PER_TILE)])

  return k


def _sc_degrees(srcp, dstp, zeros16):
  return _deg_kernel()(srcp, dstp, zeros16)


@functools.cache
def _agg_kernel():
  @functools.partial(
      pl.kernel,
      out_type=jax.ShapeDtypeStruct((NC, NP, H), _f32),
      mesh=_mesh(),
      scratch_types=[
          pltpu.VMEM((CH, CHUNK), jnp.int32),
          pltpu.VMEM((CH, CHUNK), jnp.int32),
          pltpu.VMEM((CHUNK, H), _f32),
          pltpu.VMEM((CHUNK, H), _f32),
          pltpu.VMEM((CHUNK, H), _f32),
          pltpu.VMEM_SHARED((NP, H), _f32),
          pltpu.VMEM_SHARED((NP, H), _f32),
          pltpu.SemaphoreType.DMA,
          pltpu.SemaphoreType.DMA,
          pltpu.SemaphoreType.DMA,
          pltpu.SemaphoreType.DMA,
          pltpu.SemaphoreType.DMA,
          pltpu.SemaphoreType.DMA,
      ],
      compiler_params=pltpu.CompilerParams(use_tc_tiling_on_sc=False),
  )
  def k(ytab_hbm, src_hbm, dst_hbm, zeros_hbm, out_hbm,
        idx_s, idx_d, rows_a, rows_b, rows_c, ytab_sh, agg_sh,
        sga, sgb, sgc, ssa, ssb, ssc):
    c = lax.axis_index("c")
    s = lax.axis_index("s")
    w = s * NC + c
    r0 = s * ROWS_PER_TILE

    pltpu.sync_copy(zeros_hbm.at[pl.ds(r0, ROWS_PER_TILE)],
                    agg_sh.at[pl.ds(r0, ROWS_PER_TILE)])
    pltpu.sync_copy(ytab_hbm.at[pl.ds(r0, ROWS_PER_TILE)],
                    ytab_sh.at[pl.ds(r0, ROWS_PER_TILE)])
    pltpu.sync_copy(src_hbm.at[pl.ds(w * CH, CH)], idx_s)
    pltpu.sync_copy(dst_hbm.at[pl.ds(w * CH, CH)], idx_d)
    plsc.subcore_barrier()

    bufs = ((rows_a, sga, ssa), (rows_b, sgb, ssb), (rows_c, sgc, ssc))

    def gather_start(j, b):
      pltpu.async_copy(ytab_sh.at[idx_s.at[j]], bufs[b][0], bufs[b][1])

    def gather_wait(j, b):
      pltpu.make_async_copy(ytab_sh.at[idx_s.at[j]], bufs[b][0],
                            bufs[b][1]).wait()

    def scat_start(j, b):
      pltpu.async_copy(bufs[b][0], agg_sh.at[idx_d.at[j]], bufs[b][2],
                       add=True)

    def scat_wait(j, b):
      pltpu.make_async_copy(bufs[b][0], agg_sh.at[idx_d.at[j]],
                            bufs[b][2]).wait()

    gather_start(0, 0)
    gather_start(1, 1)

    @pl.loop(0, (CH - 2) // 3)
    def _edge_chunk(jj):
      j0 = jj * 3
      for b in range(3):
        j = j0 + b
        gather_wait(j, b)
        scat_start(j, b)
        nb = (b + 2) % 3

        @pl.when(j > 0)
        def _drain():
          scat_wait(j - 1, nb)

        gather_start(j + 2, nb)

    for j, b in ((CH - 2, (CH - 2) % 3), (CH - 1, (CH - 1) % 3)):
      gather_wait(j, b)
      scat_start(j, b)
    for j in (CH - 3, CH - 2, CH - 1):
      scat_wait(j, j % 3)

    plsc.subcore_barrier()
    pltpu.sync_copy(agg_sh.at[pl.ds(r0, ROWS_PER_TILE)],
                    out_hbm.at[c, pl.ds(r0, ROWS_PER_TILE)])

  return k


def _sc_aggregate(ytab, srcp, dstp, zeros64):
  return _agg_kernel()(ytab, srcp, dstp, zeros64)


BLK = 640
NBLK = NP // BLK


def _norm_from(p0, p1):
  deg = p0[:, :1] + p1[:, :1]
  return jnp.where(deg > 0, lax.rsqrt(jnp.maximum(deg, 1.0)), 0.0)


def _tc_layer0(x, dgo0, dgo1, W0, Rw0, Rb0):
  def body(x_ref, o0_ref, o1_ref, w_ref, rw_ref, rb_ref, y_ref, res_ref):
    xb = x_ref[...]
    norm_o = _norm_from(o0_ref[...], o1_ref[...])
    y_ref[...] = jnp.dot(xb * norm_o, w_ref[...],
                         preferred_element_type=_f32)
    res_ref[...] = jax.nn.relu(
        jnp.dot(xb, rw_ref[...], preferred_element_type=_f32) + rb_ref[...])

  return pl.pallas_call(
      body,
      grid=(NBLK,),
      in_specs=[
          pl.BlockSpec((BLK, DIN), lambda k: (k, 0)),
          pl.BlockSpec((BLK, 16), lambda k: (k, 0)),
          pl.BlockSpec((BLK, 16), lambda k: (k, 0)),
          pl.BlockSpec((DIN, H), lambda k: (0, 0)),
          pl.BlockSpec((DIN, H), lambda k: (0, 0)),
          pl.BlockSpec((1, H), lambda k: (0, 0)),
      ],
      out_specs=[
          pl.BlockSpec((BLK, H), lambda k: (k, 0)),
          pl.BlockSpec((BLK, H), lambda k: (k, 0)),
      ],
      out_shape=[jax.ShapeDtypeStruct((NP, H), _f32),
                 jax.ShapeDtypeStruct((NP, H), _f32)],
  )(x, dgo0, dgo1, W0, Rw0, Rb0)


def _tc_mid(p0, p1, res_prev, dgo0, dgo1, dgi0, dgi1,
            b_prev, g_prev, beta_prev, m_prev, v_prev, Wn, Rwn, Rbn):
  def body(p0_ref, p1_ref, rp_ref, o0_ref, o1_ref, i0_ref, i1_ref,
           b_ref, g_ref, be_ref, m_ref, v_ref, w_ref, rw_ref, rb_ref,
           y_ref, res_ref):
    norm_i = _norm_from(i0_ref[...], i1_ref[...])
    h = jax.nn.relu(norm_i * (p0_ref[...] + p1_ref[...]) + b_ref[...])
    h = h + rp_ref[...]
    h = (h - m_ref[...]) * lax.rsqrt(v_ref[...] + _EPS) * g_ref[...] \
        + be_ref[...]
    norm_o = _norm_from(o0_ref[...], o1_ref[...])
    y_ref[...] = jnp.dot(h * norm_o, w_ref[...],
                         preferred_element_type=_f32)
    res_ref[...] = jax.nn.relu(
        jnp.dot(h, rw_ref[...], preferred_element_type=_f32) + rb_ref[...])

  row = lambda k: (k, 0)
  fix = lambda k: (0, 0)
  return pl.pallas_call(
      body,
      grid=(NBLK,),
      in_specs=[
          pl.BlockSpec((BLK, H), row), pl.BlockSpec((BLK, H), row),
          pl.BlockSpec((BLK, H), row),
          pl.BlockSpec((BLK, 16), row), pl.BlockSpec((BLK, 16), row),
          pl.BlockSpec((BLK, 16), row), pl.BlockSpec((BLK, 16), row),
          pl.BlockSpec((1, H), fix), pl.BlockSpec((1, H), fix),
          pl.BlockSpec((1, H), fix), pl.BlockSpec((1, H), fix),
          pl.BlockSpec((1, H), fix),
          pl.BlockSpec((H, H), fix), pl.BlockSpec((H, H), fix),
          pl.BlockSpec((1, H), fix),
      ],
      out_specs=[
          pl.BlockSpec((BLK, H), row),
          pl.BlockSpec((BLK, H), row),
      ],
      out_shape=[jax.ShapeDtypeStruct((NP, H), _f32),
                 jax.ShapeDtypeStruct((NP, H), _f32)],
  )(p0, p1, res_prev, dgo0, dgo1, dgi0, dgi1,
    b_prev, g_prev, beta_prev, m_prev, v_prev, Wn, Rwn, Rbn)


def _tc_final(p0, p1, res_prev, dgi0, dgi1,
              b2, g2, beta2, m2, v2, gid3, Wc1, bc1, Wc2, bc2):
  def body(p0_ref, p1_ref, rp_ref, i0_ref, i1_ref,
           b_ref, g_ref, be_ref, m_ref, v_ref, gid_ref,
           wc1_ref, bc1_ref, wc2_ref, bc2_ref, pooled_ref, out_ref):
    k = pl.program_id(0)
    norm_i = _norm_from(i0_ref[...], i1_ref[...])
    h = jax.nn.relu(norm_i * (p0_ref[...] + p1_ref[...]) + b_ref[...])
    h = h + rp_ref[...]
    h = (h - m_ref[...]) * lax.rsqrt(v_ref[...] + _EPS) * g_ref[...] \
        + be_ref[...]
    gid = gid_ref[0]
    seg = lax.broadcasted_iota(jnp.int32, (G, BLK), 0)
    mask = (seg == gid).astype(_f32)
    contrib = jnp.dot(mask, h, preferred_element_type=_f32)

    @pl.when(k == 0)
    def _init():
      pooled_ref[...] = contrib

    @pl.when(k > 0)
    def _acc():
      pooled_ref[...] += contrib

    @pl.when(k == NBLK - 1)
    def _head():
      pooled = pooled_ref[...]
      hid = jax.nn.relu(
          jnp.dot(pooled, wc1_ref[...], preferred_element_type=_f32)
          + bc1_ref[...])
      out_ref[...] = jnp.dot(hid, wc2_ref[...],
                             preferred_element_type=_f32) + bc2_ref[...]

  row = lambda k: (k, 0)
  fix = lambda k: (0, 0)
  return pl.pallas_call(
      body,
      grid=(NBLK,),
      in_specs=[
          pl.BlockSpec((BLK, H), row), pl.BlockSpec((BLK, H), row),
          pl.BlockSpec((BLK, H), row),
          pl.BlockSpec((BLK, 16), row), pl.BlockSpec((BLK, 16), row),
          pl.BlockSpec((1, H), fix), pl.BlockSpec((1, H), fix),
          pl.BlockSpec((1, H), fix), pl.BlockSpec((1, H), fix),
          pl.BlockSpec((1, H), fix),
          pl.BlockSpec((1, 1, BLK), lambda k: (k, 0, 0)),
          pl.BlockSpec((H, G), fix), pl.BlockSpec((1, G), fix),
          pl.BlockSpec((G, 2), fix), pl.BlockSpec((1, 2), fix),
      ],
      out_specs=[
          pl.BlockSpec((G, H), fix),
          pl.BlockSpec((G, 2), fix),
      ],
      out_shape=[jax.ShapeDtypeStruct((G, H), _f32),
                 jax.ShapeDtypeStruct((G, 2), _f32)],
  )(p0, p1, res_prev, dgi0, dgi1,
    b2, g2, beta2, m2, v2, gid3, Wc1, bc1, Wc2, bc2)[1]


def kernel(node_feats, edge_index, graph_ids,
           W0, b0, Rw0, Rb0, g0, beta0, m0, v0,
           W1, b1, Rw1, Rb1, g1, beta1, m1, v1,
           W2, b2, Rw2, Rb2, g2, beta2, m2, v2,
           Wc1, bc1, Wc2, bc2):
  x = jnp.pad(node_feats, ((0, NP - N), (0, 0)))
  pad_e = EP - E
  srcp = jnp.concatenate(
      [edge_index[0], jnp.full((pad_e,), N, jnp.int32)]).reshape(NW * CH, CHUNK)
  dstp = jnp.concatenate(
      [edge_index[1], jnp.full((pad_e,), N, jnp.int32)]).reshape(NW * CH, CHUNK)
  gid3 = jnp.pad(graph_ids, (0, NP - N),
                 constant_values=G + 1).reshape(NBLK, 1, BLK)
  zeros64 = jnp.zeros((NP, H), _f32)
  zeros16 = jnp.zeros((NP, 16), _f32)
  as_row = lambda t: t.reshape(1, -1)

  dego, degi = _sc_degrees(srcp, dstp, zeros16)
  dgo0, dgo1 = dego[0], dego[1]
  dgi0, dgi1 = degi[0], degi[1]

  y0, res0 = _tc_layer0(x, dgo0, dgo1, W0, Rw0, as_row(Rb0))
  p = _sc_aggregate(y0, srcp, dstp, zeros64)

  y1, res1 = _tc_mid(p[0], p[1], res0, dgo0, dgo1, dgi0, dgi1,
                     as_row(b0), as_row(g0), as_row(beta0), as_row(m0),
                     as_row(v0), W1, Rw1, as_row(Rb1))
  p = _sc_aggregate(y1, srcp, dstp, zeros64)

  y2, res2 = _tc_mid(p[0], p[1], res1, dgo0, dgo1, dgi0, dgi1,
                     as_row(b1), as_row(g1), as_row(beta1), as_row(m1),
                     as_row(v1), W2, Rw2, as_row(Rb2))
  p = _sc_aggregate(y2, srcp, dstp, zeros64)

  return _tc_final(p[0], p[1], res2, dgi0, dgi1,
                   as_row(b2), as_row(g2), as_row(beta2), as_row(m2),
                   as_row(v2), gid3, Wc1, bc1.reshape(1, G), Wc2,
                   bc2.reshape(1, 2))


# speedup vs baseline: 1.0956x; 1.0082x over previous
import functools

import jax
import jax.numpy as jnp
from jax import lax
from jax.experimental import pallas as pl
from jax.experimental.pallas import tpu as pltpu
from jax.experimental.pallas import tpu_sc as plsc

N = 10000
NP = 10240
E = 320000
G = 128
DIN = 128
H = 64
NC, NS, L = 2, 16, 16
NW = NC * NS
CH = 80
EP = NW * CH * 128
CHUNK = 128
ROWS_PER_TILE = NP // NS

_EPS = 1e-5
_f32 = jnp.float32


def _mesh():
  return plsc.VectorSubcoreMesh(core_axis_name="c", subcore_axis_name="s",
                                num_cores=NC, num_subcores=NS)


def _fill_const(buf, width, value):
  @pl.loop(0, CHUNK)
  def _fill(i):
    for kk in range(width // L):
      buf[i, pl.ds(kk * L, L)] = jnp.full((L,), value, _f32)


@functools.cache
def _deg_kernel():
  @functools.partial(
      pl.kernel,
      out_type=[jax.ShapeDtypeStruct((NC, NP, 16), _f32),
                jax.ShapeDtypeStruct((NC, NP, 16), _f32)],
      mesh=_mesh(),
      scratch_types=[
          pltpu.VMEM((CH, CHUNK), jnp.int32),
          pltpu.VMEM((CH, CHUNK), jnp.int32),
          pltpu.VMEM((CHUNK, 16), _f32),
          pltpu.VMEM_SHARED((NP, 16), _f32),
          pltpu.VMEM_SHARED((NP, 16), _f32),
          pltpu.SemaphoreType.DMA,
      ],
      compiler_params=pltpu.CompilerParams(use_tc_tiling_on_sc=False),
  )
  def k(src_hbm, dst_hbm, zeros_hbm, dego_hbm, degi_hbm,
        idx_s, idx_d, ones_v, dego_sh, degi_sh, sem):
    c = lax.axis_index("c")
    s = lax.axis_index("s")
    w = s * NC + c
    r0 = s * ROWS_PER_TILE

    _fill_const(ones_v, 16, 1.0)

    pltpu.sync_copy(zeros_hbm.at[pl.ds(r0, ROWS_PER_TILE)],
                    dego_sh.at[pl.ds(r0, ROWS_PER_TILE)])
    pltpu.sync_copy(zeros_hbm.at[pl.ds(r0, ROWS_PER_TILE)],
                    degi_sh.at[pl.ds(r0, ROWS_PER_TILE)])
    pltpu.sync_copy(src_hbm.at[pl.ds(w * CH, CH)], idx_s)
    pltpu.sync_copy(dst_hbm.at[pl.ds(w * CH, CH)], idx_d)
    plsc.subcore_barrier()

    @pl.loop(0, CH // 8)
    def _scatter(t):
      j0 = t * 8
      for u in range(8):
        pltpu.async_copy(ones_v, dego_sh.at[idx_s.at[j0 + u]], sem,
                         add=True)
        pltpu.async_copy(ones_v, degi_sh.at[idx_d.at[j0 + u]], sem,
                         add=True)
      for u in range(8):
        pltpu.make_async_copy(ones_v, dego_sh.at[idx_s.at[j0 + u]],
                              sem).wait()
        pltpu.make_async_copy(ones_v, degi_sh.at[idx_d.at[j0 + u]],
                              sem).wait()

    plsc.subcore_barrier()
    pltpu.sync_copy(dego_sh.at[pl.ds(r0, ROWS_PER_TILE)],
                    dego_hbm.at[c, pl.ds(r0, ROWS_PER_TILE)])
    pltpu.sync_copy(degi_sh.at[pl.ds(r0, ROWS_PER_TILE)],
                    degi_hbm.at[c, pl.ds(r0, ROWS_PER_TILE)])

  return k


def _sc_degrees(srcp, dstp, zeros16):
  return _deg_kernel()(srcp, dstp, zeros16)


@functools.cache
def _agg_kernel():
  @functools.partial(
      pl.kernel,
      out_type=jax.ShapeDtypeStruct((NC, NP, H), _f32),
      mesh=_mesh(),
      scratch_types=[
          pltpu.VMEM((CH, CHUNK), jnp.int32),
          pltpu.VMEM((CH, CHUNK), jnp.int32),
          pltpu.VMEM((CHUNK, H), _f32),
          pltpu.VMEM((CHUNK, H), _f32),
          pltpu.VMEM((CHUNK, H), _f32),
          pltpu.VMEM_SHARED((NP, H), _f32),
          pltpu.VMEM_SHARED((NP, H), _f32),
          pltpu.SemaphoreType.DMA,
          pltpu.SemaphoreType.DMA,
          pltpu.SemaphoreType.DMA,
          pltpu.SemaphoreType.DMA,
          pltpu.SemaphoreType.DMA,
          pltpu.SemaphoreType.DMA,
      ],
      compiler_params=pltpu.CompilerParams(use_tc_tiling_on_sc=False),
  )
  def k(ytab_hbm, src_hbm, dst_hbm, zeros_hbm, out_hbm,
        idx_s, idx_d, rows_a, rows_b, rows_c, ytab_sh, agg_sh,
        sga, sgb, sgc, ssa, ssb, ssc):
    c = lax.axis_index("c")
    s = lax.axis_index("s")
    w = s * NC + c
    r0 = s * ROWS_PER_TILE

    stage = (
        (zeros_hbm.at[pl.ds(r0, ROWS_PER_TILE)],
         agg_sh.at[pl.ds(r0, ROWS_PER_TILE)]),
        (ytab_hbm.at[pl.ds(r0, ROWS_PER_TILE)],
         ytab_sh.at[pl.ds(r0, ROWS_PER_TILE)]),
        (src_hbm.at[pl.ds(w * CH, CH)], idx_s),
        (dst_hbm.at[pl.ds(w * CH, CH)], idx_d),
    )
    for src, dst in stage:
      pltpu.async_copy(src, dst, sga)
    for src, dst in stage:
      pltpu.make_async_copy(src, dst, sga).wait()
    plsc.subcore_barrier()

    bufs = ((rows_a, sga, ssa), (rows_b, sgb, ssb), (rows_c, sgc, ssc))

    def gather_start(j, b):
      pltpu.async_copy(ytab_sh.at[idx_s.at[j]], bufs[b][0], bufs[b][1])

    def gather_wait(j, b):
      pltpu.make_async_copy(ytab_sh.at[idx_s.at[j]], bufs[b][0],
                            bufs[b][1]).wait()

    def scat_start(j, b):
      pltpu.async_copy(bufs[b][0], agg_sh.at[idx_d.at[j]], bufs[b][2],
                       add=True)

    def scat_wait(j, b):
      pltpu.make_async_copy(bufs[b][0], agg_sh.at[idx_d.at[j]],
                            bufs[b][2]).wait()

    gather_start(0, 0)
    gather_start(1, 1)

    @pl.loop(0, (CH - 2) // 3)
    def _edge_chunk(jj):
      j0 = jj * 3
      for b in range(3):
        j = j0 + b
        gather_wait(j, b)
        scat_start(j, b)
        nb = (b + 2) % 3

        @pl.when(j > 0)
        def _drain():
          scat_wait(j - 1, nb)

        gather_start(j + 2, nb)

    for j, b in ((CH - 2, (CH - 2) % 3), (CH - 1, (CH - 1) % 3)):
      gather_wait(j, b)
      scat_start(j, b)
    for j in (CH - 3, CH - 2, CH - 1):
      scat_wait(j, j % 3)

    plsc.subcore_barrier()
    pltpu.sync_copy(agg_sh.at[pl.ds(r0, ROWS_PER_TILE)],
                    out_hbm.at[c, pl.ds(r0, ROWS_PER_TILE)])

  return k


def _sc_aggregate(ytab, srcp, dstp, zeros64):
  return _agg_kernel()(ytab, srcp, dstp, zeros64)


BLK = 640
NBLK = NP // BLK


def _norm_from(p0, p1):
  deg = p0[:, :1] + p1[:, :1]
  return jnp.where(deg > 0, lax.rsqrt(jnp.maximum(deg, 1.0)), 0.0)


def _tc_layer0(x, dgo0, dgo1, W0, Rw0, Rb0):
  def body(x_ref, o0_ref, o1_ref, w_ref, rw_ref, rb_ref, y_ref, res_ref):
    xb = x_ref[...]
    norm_o = _norm_from(o0_ref[...], o1_ref[...])
    y_ref[...] = jnp.dot(xb * norm_o, w_ref[...],
                         preferred_element_type=_f32)
    res_ref[...] = jax.nn.relu(
        jnp.dot(xb, rw_ref[...], preferred_element_type=_f32) + rb_ref[...])

  return pl.pallas_call(
      body,
      grid=(NBLK,),
      in_specs=[
          pl.BlockSpec((BLK, DIN), lambda k: (k, 0)),
          pl.BlockSpec((BLK, 16), lambda k: (k, 0)),
          pl.BlockSpec((BLK, 16), lambda k: (k, 0)),
          pl.BlockSpec((DIN, H), lambda k: (0, 0)),
          pl.BlockSpec((DIN, H), lambda k: (0, 0)),
          pl.BlockSpec((1, H), lambda k: (0, 0)),
      ],
      out_specs=[
          pl.BlockSpec((BLK, H), lambda k: (k, 0)),
          pl.BlockSpec((BLK, H), lambda k: (k, 0)),
      ],
      out_shape=[jax.ShapeDtypeStruct((NP, H), _f32),
                 jax.ShapeDtypeStruct((NP, H), _f32)],
  )(x, dgo0, dgo1, W0, Rw0, Rb0)


def _tc_mid(p0, p1, res_prev, dgo0, dgo1, dgi0, dgi1,
            b_prev, g_prev, beta_prev, m_prev, v_prev, Wn, Rwn, Rbn):
  def body(p0_ref, p1_ref, rp_ref, o0_ref, o1_ref, i0_ref, i1_ref,
           b_ref, g_ref, be_ref, m_ref, v_ref, w_ref, rw_ref, rb_ref,
           y_ref, res_ref):
    norm_i = _norm_from(i0_ref[...], i1_ref[...])
    h = jax.nn.relu(norm_i * (p0_ref[...] + p1_ref[...]) + b_ref[...])
    h = h + rp_ref[...]
    h = (h - m_ref[...]) * lax.rsqrt(v_ref[...] + _EPS) * g_ref[...] \
        + be_ref[...]
    norm_o = _norm_from(o0_ref[...], o1_ref[...])
    y_ref[...] = jnp.dot(h * norm_o, w_ref[...],
                         preferred_element_type=_f32)
    res_ref[...] = jax.nn.relu(
        jnp.dot(h, rw_ref[...], preferred_element_type=_f32) + rb_ref[...])

  row = lambda k: (k, 0)
  fix = lambda k: (0, 0)
  return pl.pallas_call(
      body,
      grid=(NBLK,),
      in_specs=[
          pl.BlockSpec((BLK, H), row), pl.BlockSpec((BLK, H), row),
          pl.BlockSpec((BLK, H), row),
          pl.BlockSpec((BLK, 16), row), pl.BlockSpec((BLK, 16), row),
          pl.BlockSpec((BLK, 16), row), pl.BlockSpec((BLK, 16), row),
          pl.BlockSpec((1, H), fix), pl.BlockSpec((1, H), fix),
          pl.BlockSpec((1, H), fix), pl.BlockSpec((1, H), fix),
          pl.BlockSpec((1, H), fix),
          pl.BlockSpec((H, H), fix), pl.BlockSpec((H, H), fix),
          pl.BlockSpec((1, H), fix),
      ],
      out_specs=[
          pl.BlockSpec((BLK, H), row),
          pl.BlockSpec((BLK, H), row),
      ],
      out_shape=[jax.ShapeDtypeStruct((NP, H), _f32),
                 jax.ShapeDtypeStruct((NP, H), _f32)],
  )(p0, p1, res_prev, dgo0, dgo1, dgi0, dgi1,
    b_prev, g_prev, beta_prev, m_prev, v_prev, Wn, Rwn, Rbn)


def _tc_final(p0, p1, res_prev, dgi0, dgi1,
              b2, g2, beta2, m2, v2, gid3, Wc1, bc1, Wc2, bc2):
  def body(p0_ref, p1_ref, rp_ref, i0_ref, i1_ref,
           b_ref, g_ref, be_ref, m_ref, v_ref, gid_ref,
           wc1_ref, bc1_ref, wc2_ref, bc2_ref, pooled_ref, out_ref):
    k = pl.program_id(0)
    norm_i = _norm_from(i0_ref[...], i1_ref[...])
    h = jax.nn.relu(norm_i * (p0_ref[...] + p1_ref[...]) + b_ref[...])
    h = h + rp_ref[...]
    h = (h - m_ref[...]) * lax.rsqrt(v_ref[...] + _EPS) * g_ref[...] \
        + be_ref[...]
    gid = gid_ref[0]
    seg = lax.broadcasted_iota(jnp.int32, (G, BLK), 0)
    mask = (seg == gid).astype(_f32)
    contrib = jnp.dot(mask, h, preferred_element_type=_f32)

    @pl.when(k == 0)
    def _init():
      pooled_ref[...] = contrib

    @pl.when(k > 0)
    def _acc():
      pooled_ref[...] += contrib

    @pl.when(k == NBLK - 1)
    def _head():
      pooled = pooled_ref[...]
      hid = jax.nn.relu(
          jnp.dot(pooled, wc1_ref[...], preferred_element_type=_f32)
          + bc1_ref[...])
      out_ref[...] = jnp.dot(hid, wc2_ref[...],
                             preferred_element_type=_f32) + bc2_ref[...]

  row = lambda k: (k, 0)
  fix = lambda k: (0, 0)
  return pl.pallas_call(
      body,
      grid=(NBLK,),
      in_specs=[
          pl.BlockSpec((BLK, H), row), pl.BlockSpec((BLK, H), row),
          pl.BlockSpec((BLK, H), row),
          pl.BlockSpec((BLK, 16), row), pl.BlockSpec((BLK, 16), row),
          pl.BlockSpec((1, H), fix), pl.BlockSpec((1, H), fix),
          pl.BlockSpec((1, H), fix), pl.BlockSpec((1, H), fix),
          pl.BlockSpec((1, H), fix),
          pl.BlockSpec((1, 1, BLK), lambda k: (k, 0, 0)),
          pl.BlockSpec((H, G), fix), pl.BlockSpec((1, G), fix),
          pl.BlockSpec((G, 2), fix), pl.BlockSpec((1, 2), fix),
      ],
      out_specs=[
          pl.BlockSpec((G, H), fix),
          pl.BlockSpec((G, 2), fix),
      ],
      out_shape=[jax.ShapeDtypeStruct((G, H), _f32),
                 jax.ShapeDtypeStruct((G, 2), _f32)],
  )(p0, p1, res_prev, dgi0, dgi1,
    b2, g2, beta2, m2, v2, gid3, Wc1, bc1, Wc2, bc2)[1]


def kernel(node_feats, edge_index, graph_ids,
           W0, b0, Rw0, Rb0, g0, beta0, m0, v0,
           W1, b1, Rw1, Rb1, g1, beta1, m1, v1,
           W2, b2, Rw2, Rb2, g2, beta2, m2, v2,
           Wc1, bc1, Wc2, bc2):
  x = jnp.pad(node_feats, ((0, NP - N), (0, 0)))
  pad_e = EP - E
  srcp = jnp.concatenate(
      [edge_index[0], jnp.full((pad_e,), N, jnp.int32)]).reshape(NW * CH, CHUNK)
  dstp = jnp.concatenate(
      [edge_index[1], jnp.full((pad_e,), N, jnp.int32)]).reshape(NW * CH, CHUNK)
  gid3 = jnp.pad(graph_ids, (0, NP - N),
                 constant_values=G + 1).reshape(NBLK, 1, BLK)
  zeros64 = jnp.zeros((NP, H), _f32)
  zeros16 = jnp.zeros((NP, 16), _f32)
  as_row = lambda t: t.reshape(1, -1)

  dego, degi = _sc_degrees(srcp, dstp, zeros16)
  dgo0, dgo1 = dego[0], dego[1]
  dgi0, dgi1 = degi[0], degi[1]

  y0, res0 = _tc_layer0(x, dgo0, dgo1, W0, Rw0, as_row(Rb0))
  p = _sc_aggregate(y0, srcp, dstp, zeros64)

  y1, res1 = _tc_mid(p[0], p[1], res0, dgo0, dgo1, dgi0, dgi1,
                     as_row(b0), as_row(g0), as_row(beta0), as_row(m0),
                     as_row(v0), W1, Rw1, as_row(Rb1))
  p = _sc_aggregate(y1, srcp, dstp, zeros64)

  y2, res2 = _tc_mid(p[0], p[1], res1, dgo0, dgo1, dgi0, dgi1,
                     as_row(b1), as_row(g1), as_row(beta1), as_row(m1),
                     as_row(v1), W2, Rw2, as_row(Rb2))
  p = _sc_aggregate(y2, srcp, dstp, zeros64)

  return _tc_final(p[0], p[1], res2, dgi0, dgi1,
                   as_row(b2), as_row(g2), as_row(beta2), as_row(m2),
                   as_row(v2), gid3, Wc1, bc1.reshape(1, G), Wc2,
                   bc2.reshape(1, 2))
